# baseline (speedup 1.0000x reference)
import functools

import jax
import jax.numpy as jnp
import numpy as np
from jax import lax
from jax.experimental import pallas as pl
from jax.experimental.pallas import tpu as pltpu
from jax.experimental.pallas import tpu_sc as plsc

N_NODES = 10000
N_EDGES = 160000
B = 8
HEADS = 16
SIZE = 500
NPAD = 10240
NT = 32
NPT = 320
QCAP = 12288
QW = QCAP * 4
CH = 128
SCH = 4000
NEG = -1e30
SENT = float(NPAD - 1)

_SC_PARAMS = pltpu.CompilerParams(
    use_tc_tiling_on_sc=False, needs_layout_passes=False)

_f32 = jnp.float32
_i32 = jnp.int32

_TMAP = np.array([hd * 5 + ch for ch in range(5) for hd in range(16)])
_NTMAP = np.array([ch * 16 + hd for hd in range(16) for ch in range(5)])


def _mesh():
    return plsc.VectorSubcoreMesh(core_axis_name="c", subcore_axis_name="s")


def _prefix16(csv, iota):
    for sh in (1, 2, 4, 8):
        sht = csv[jnp.maximum(iota - sh, 0)]
        csv = csv + jnp.where(iota >= sh, sht, 0.0)
    return csv


def _bucket_body(src_hbm, dst_hbm, e0_hbm, e1_hbm,
                 q_hbm, cnt_hbm,
                 sbuf, dbuf, e0b, e1b, qw, ob, sq):
    wid = lax.axis_index("s") * 2 + lax.axis_index("c")
    iota = lax.broadcasted_iota(_i32, (16,), 0)
    basef = (wid * NPT).astype(_f32)
    nchunks = N_EDGES // SCH
    sentv = jnp.full((16,), SENT, _f32)
    basev = jnp.full((16,), 0.0, _f32) + basef
    zerov = jnp.zeros((16,), _f32)

    def initq(i, _):
        fld = (i >> 3) & 3
        v = jnp.where(fld == 0, sentv, jnp.where(fld == 1, basev, zerov))
        qw[pl.ds(i * 16, 16)] = v
        return 0
    lax.fori_loop(0, QW // 16, initq, 0)

    widv = jnp.full((16,), 0, _i32) + wid

    def fire(c, par):
        off = c * SCH
        hof = par * SCH
        pltpu.async_copy(src_hbm.at[pl.ds(off, SCH)],
                         sbuf.at[pl.ds(hof, SCH)], sq)
        pltpu.async_copy(dst_hbm.at[pl.ds(off, SCH)],
                         dbuf.at[pl.ds(hof, SCH)], sq)
        pltpu.async_copy(e0_hbm.at[pl.ds(off, SCH)],
                         e0b.at[pl.ds(hof, SCH)], sq)
        pltpu.async_copy(e1_hbm.at[pl.ds(off, SCH)],
                         e1b.at[pl.ds(hof, SCH)], sq)

    def drain(c, par):
        off = c * SCH
        hof = par * SCH
        pltpu.make_async_copy(src_hbm.at[pl.ds(off, SCH)],
                              sbuf.at[pl.ds(hof, SCH)], sq).wait()
        pltpu.make_async_copy(dst_hbm.at[pl.ds(off, SCH)],
                              dbuf.at[pl.ds(hof, SCH)], sq).wait()
        pltpu.make_async_copy(e0_hbm.at[pl.ds(off, SCH)],
                              e0b.at[pl.ds(hof, SCH)], sq).wait()
        pltpu.make_async_copy(e1_hbm.at[pl.ds(off, SCH)],
                              e1b.at[pl.ds(hof, SCH)], sq).wait()

    fire(0, 0)

    def chunk(c, cnt):
        par = c & 1
        drain(c, par)

        @pl.when(c + 1 < nchunks)
        def _():
            fire(c + 1, 1 - par)

        hof = par * SCH

        def grp(g, cnt):
            dv = dbuf[pl.ds(hof + g * 16, 16)]
            bkt = (dv * 52429) >> 24
            own = bkt == widv
            mf = jnp.where(own, 1.0, 0.0)
            cs = _prefix16(mf, iota)
            posf = cnt + (cs - mf)
            posf = jnp.minimum(posf, float(QCAP - CH - 16))
            pos = posf.astype(_i32)
            qpos = ((pos >> 7) * 512) + (pos & 127)
            trash = (QCAP // CH - 1) * 512 + iota
            qpos = jnp.where(own, qpos, trash)
            sv = sbuf[pl.ds(hof + g * 16, 16)].astype(_f32)
            plsc.store_scatter(qw, [qpos], sv)
            plsc.store_scatter(qw, [qpos + 128], dv.astype(_f32))
            plsc.store_scatter(qw, [qpos + 256], e0b[pl.ds(hof + g * 16, 16)])
            plsc.store_scatter(qw, [qpos + 384], e1b[pl.ds(hof + g * 16, 16)])
            return cnt + cs[15]
        return lax.fori_loop(0, SCH // 16, grp, cnt)

    cnt = lax.fori_loop(0, nchunks, chunk, jnp.float32(0.0))
    cnt = jnp.minimum(cnt, float(QCAP - CH - 16))
    toff = (QCAP // CH - 1) * 512

    def fixt(i, _):
        fld = i >> 3
        v = jnp.where(fld == 0, sentv, jnp.where(fld == 1, basev, zerov))
        qw[pl.ds(toff + i * 16, 16)] = v
        return 0
    lax.fori_loop(0, 512 // 16, fixt, 0)

    ob[...] = jnp.full((16,), 0.0, _f32) + cnt
    pltpu.sync_copy(ob, cnt_hbm.at[wid])
    pltpu.sync_copy(qw, q_hbm.at[wid])


def _bucket(src, dst, ev0, ev1):
    kfn = pl.kernel(
        _bucket_body,
        out_type=[jax.ShapeDtypeStruct((NT, QW), _f32),
                  jax.ShapeDtypeStruct((NT, 16), _f32)],
        mesh=_mesh(),
        compiler_params=_SC_PARAMS,
        scratch_types=[
            pltpu.VMEM((2 * SCH,), _i32), pltpu.VMEM((2 * SCH,), _i32),
            pltpu.VMEM((2 * SCH,), _f32), pltpu.VMEM((2 * SCH,), _f32),
            pltpu.VMEM((QW,), _f32),
            pltpu.VMEM((16,), _f32),
            pltpu.SemaphoreType.DMA,
        ],
    )
    return kfn(src, dst, ev0, ev1)


def _gat_body(masked, h2d, asd2d, add2d, mef, biasf,
              q_hbm, cnt_hbm, out_hbm,
              m_t, den_t, acc_t, addl, asdl, hl,
              qb, idxc, a16, h80, meb, biasb, cb,
              sq, s1, s2):
    wid = lax.axis_index("s") * 2 + lax.axis_index("c")
    iota = lax.broadcasted_iota(_i32, (16,), 0)
    base = wid * NPT
    basef = jnp.full((16,), 0.0, _f32) + base.astype(_f32)

    pltpu.sync_copy(cnt_hbm.at[wid], cb)
    nmy = cb[...][0].astype(_i32)
    pltpu.sync_copy(add2d.at[pl.ds(base, NPT)], addl)
    pltpu.sync_copy(asd2d.at[pl.ds(base, NPT)], asdl)
    pltpu.sync_copy(h2d.at[pl.ds(base, NPT)], hl)
    pltpu.sync_copy(mef, meb)
    pltpu.sync_copy(biasf, biasb)
    me0 = meb[pl.ds(0, 16)]
    me1 = meb[pl.ds(16, 16)]

    negv = jnp.full((16,), NEG, _f32)
    zerov = jnp.zeros((16,), _f32)

    def init1(i, _):
        m_t[pl.ds(i * 16, 16)] = negv
        den_t[pl.ds(i * 16, 16)] = zerov
        return 0
    lax.fori_loop(0, NPT, init1, 0)

    def init2(i, _):
        acc_t[pl.ds(i * 16, 16)] = zerov
        return 0
    lax.fori_loop(0, NPT * 5, init2, 0)

    nch = (nmy + (CH - 1)) // CH

    def fire_q(c, par):
        pltpu.async_copy(q_hbm.at[wid, pl.ds(c * 512, 512)],
                         qb.at[pl.ds(par * 512, 512)], sq)

    def drain_q(c, par):
        pltpu.make_async_copy(q_hbm.at[wid, pl.ds(c * 512, 512)],
                              qb.at[pl.ds(par * 512, 512)], sq).wait()

    def cvt_fire(c, par, with_h):
        qoff = par * 512

        def cvt(g, _2):
            idxc[pl.ds(par * CH + g * 16, 16)] = \
                qb[pl.ds(qoff + g * 16, 16)].astype(_i32)
            return 0
        lax.fori_loop(0, CH // 16, cvt, 0)
        pltpu.async_copy(asd2d.at[idxc.at[pl.ds(par * CH, CH)]],
                         a16.at[pl.ds(par * CH, CH)], s1)
        if with_h:
            pltpu.async_copy(h2d.at[idxc.at[pl.ds(par * CH, CH)]],
                             h80.at[pl.ds(par * CH, CH)], s2)

    def drain_ind(par, with_h):
        pltpu.make_async_copy(asd2d.at[idxc.at[pl.ds(par * CH, CH)]],
                              a16.at[pl.ds(par * CH, CH)], s1).wait()
        if with_h:
            pltpu.make_async_copy(h2d.at[idxc.at[pl.ds(par * CH, CH)]],
                                  h80.at[pl.ds(par * CH, CH)], s2).wait()

    @pl.when(nch > 0)
    def _():
        fire_q(0, 0)
        drain_q(0, 0)
        cvt_fire(0, 0, False)

    @pl.when(nch > 1)
    def _():
        fire_q(1, 1)

    def chunk_p1(c, _):
        par = c & 1
        parn = 1 - par

        @pl.when(c + 1 < nch)
        def _():
            drain_q(c + 1, parn)
            cvt_fire(c + 1, parn, False)

        drain_ind(par, False)
        qoff = par * 512

        def grp(g, _2):
            dlv = qb[pl.ds(qoff + 128 + g * 16, 16)] - basef
            e0v = qb[pl.ds(qoff + 256 + g * 16, 16)]
            e1v = qb[pl.ds(qoff + 384 + g * 16, 16)]
            sv = qb[pl.ds(qoff + g * 16, 16)]
            ebv = jnp.where(sv == SENT, NEG, 0.0)
            for j in range(16):
                jf = jnp.full((16,), j, _i32)
                dl_i = dlv[j].astype(_i32)
                e0j = e0v[jf]
                e1j = e1v[jf]
                ebj = ebv[jf]
                fidx = dl_i * 16 + iota
                ae = e0j * me0 + e1j * me1
                if masked:
                    ae = ae + jnp.where(e0j > 1.0, 0.0, NEG)
                ae = ae + ebj
                asdr = a16[par * CH + g * 16 + j, :]
                addr = addl[dl_i, :]
                z = asdr + addr + ae
                z = jnp.maximum(z, 0.2 * z)
                mo = plsc.load_gather(m_t, [fidx])
                plsc.store_scatter(m_t, [fidx], jnp.maximum(mo, z))
            return 0
        lax.fori_loop(0, CH // 16, grp, 0)

        @pl.when(c + 2 < nch)
        def _():
            fire_q(c + 2, par)
        return 0
    lax.fori_loop(0, nch, chunk_p1, 0)

    def mself(i, _):
        asdr = asdl[i, :]
        addr = addl[i, :]
        zs = asdr + addr
        zs = jnp.maximum(zs, 0.2 * zs)
        mo = m_t[pl.ds(i * 16, 16)]
        m_t[pl.ds(i * 16, 16)] = jnp.maximum(mo, zs)
        return 0
    lax.fori_loop(0, NPT, mself, 0)

    @pl.when(nch > 0)
    def _():
        fire_q(0, 0)
        drain_q(0, 0)
        cvt_fire(0, 0, True)

    @pl.when(nch > 1)
    def _():
        fire_q(1, 1)

    def chunk_p2(c, _):
        par = c & 1
        parn = 1 - par

        @pl.when(c + 1 < nch)
        def _():
            drain_q(c + 1, parn)
            cvt_fire(c + 1, parn, True)

        drain_ind(par, True)
        qoff = par * 512

        def grp(g, _2):
            dlv = qb[pl.ds(qoff + 128 + g * 16, 16)] - basef
            e0v = qb[pl.ds(qoff + 256 + g * 16, 16)]
            e1v = qb[pl.ds(qoff + 384 + g * 16, 16)]
            sv = qb[pl.ds(qoff + g * 16, 16)]
            ebv = jnp.where(sv == SENT, NEG, 0.0)
            for j in range(16):
                jf = jnp.full((16,), j, _i32)
                dl_i = dlv[j].astype(_i32)
                e0j = e0v[jf]
                e1j = e1v[jf]
                ebj = ebv[jf]
                fidx = dl_i * 16 + iota
                ae = e0j * me0 + e1j * me1
                if masked:
                    ae = ae + jnp.where(e0j > 1.0, 0.0, NEG)
                ae = ae + ebj
                asdr = a16[par * CH + g * 16 + j, :]
                addr = addl[dl_i, :]
                z = asdr + addr + ae
                z = jnp.maximum(z, 0.2 * z)
                mo = plsc.load_gather(m_t, [fidx])
                ex = jnp.exp(z - mo)
                plsc.addupdate_scatter(den_t, [fidx], ex)
                fidx80 = dl_i * 80 + iota
                for cc in range(5):
                    fi = fidx80 + 16 * cc
                    hr = h80[par * CH + g * 16 + j, pl.ds(16 * cc, 16)]
                    plsc.addupdate_scatter(acc_t, [fi], ex * hr)
            return 0
        lax.fori_loop(0, CH // 16, grp, 0)

        @pl.when(c + 2 < nch)
        def _():
            fire_q(c + 2, par)
        return 0
    lax.fori_loop(0, nch, chunk_p2, 0)

    def fin(i, _):
        asdr = asdl[i, :]
        addr = addl[i, :]
        zs = asdr + addr
        zs = jnp.maximum(zs, 0.2 * zs)
        mo = m_t[pl.ds(i * 16, 16)]
        exs = jnp.exp(zs - mo)
        den = den_t[pl.ds(i * 16, 16)] + exs
        inv = 1.0 / (den + 1e-16)
        for cc in range(5):
            ac = acc_t[pl.ds(i * 80 + 16 * cc, 16)]
            hc = hl[i, pl.ds(16 * cc, 16)]
            acc_t[pl.ds(i * 80 + 16 * cc, 16)] = (
                (ac + exs * hc) * inv + biasb[pl.ds(16 * cc, 16)])
        return 0
    lax.fori_loop(0, NPT, fin, 0)

    pltpu.sync_copy(acc_t, out_hbm.at[pl.ds(base * 80, NPT * 80)])


def _gat_sc(h2d, asd2d, add2d, mef, biasf, queues, masked):
    qu, cnt = queues
    kfn = pl.kernel(
        functools.partial(_gat_body, masked),
        out_type=jax.ShapeDtypeStruct((NPAD * 80,), _f32),
        mesh=_mesh(),
        compiler_params=_SC_PARAMS,
        scratch_types=[
            pltpu.VMEM((NPT * 16,), _f32),
            pltpu.VMEM((NPT * 16,), _f32),
            pltpu.VMEM((NPT * 80,), _f32),
            pltpu.VMEM((NPT, 16), _f32),
            pltpu.VMEM((NPT, 16), _f32),
            pltpu.VMEM((NPT, 80), _f32),
            pltpu.VMEM((2 * 512,), _f32),
            pltpu.VMEM((2 * CH,), _i32),
            pltpu.VMEM((2 * CH, 16), _f32),
            pltpu.VMEM((2 * CH, 80), _f32),
            pltpu.VMEM((32,), _f32),
            pltpu.VMEM((80,), _f32),
            pltpu.VMEM((16,), _f32),
            pltpu.SemaphoreType.DMA,
            pltpu.SemaphoreType.DMA,
            pltpu.SemaphoreType.DMA,
        ],
    )
    return kfn(h2d, asd2d, add2d, mef, biasf, qu, cnt)


NROW = SIZE + 1


def _pool_body(xf_hbm, tf_hbm, mx_hbm, sm_hbm, cn_hbm,
               mxt, smt, cnt, xb, tb):
    wid = lax.axis_index("s") * 2 + lax.axis_index("c")
    iota = lax.broadcasted_iota(_i32, (16,), 0)
    b = wid >> 2
    r = wid & 3
    start = b * 1250 + r * 313
    nmine = jnp.where(r == 3, 311, 313)
    sstart = (start // 8) * 8
    doff = start - sstart

    lowv = jnp.full((16,), -3.0e38, _f32)
    zerov = jnp.zeros((16,), _f32)

    def init1(i, _):
        mxt[pl.ds(i * 16, 16)] = lowv
        smt[pl.ds(i * 16, 16)] = zerov
        return 0
    lax.fori_loop(0, NROW * 5, init1, 0)

    def init2(i, _):
        cnt[pl.ds(i * 16, 16)] = zerov
        return 0
    lax.fori_loop(0, NROW, init2, 0)

    pltpu.sync_copy(xf_hbm.at[pl.ds(start * 80, 313 * 80)], xb)
    pltpu.sync_copy(tf_hbm.at[pl.ds(sstart, 320)], tb)

    def grpn(g, _):
        gl = g * 16 + iota
        tv = tb[pl.ds(g * 16 + doff, 16)]
        tv = jnp.where(gl < nmine, tv, float(SIZE))
        for j in range(16):
            jf = jnp.full((16,), j, _i32)
            tj = tv[jf]
            ci80 = (tj * 80.0).astype(_i32) + iota
            ci16 = (tj * 16.0).astype(_i32) + iota
            node = g * 16 + j
            for cc in range(5):
                fi = ci80 + 16 * cc
                xc = xb[pl.ds(node * 80 + 16 * cc, 16)]
                mo = plsc.load_gather(mxt, [fi])
                plsc.store_scatter(mxt, [fi], jnp.maximum(mo, xc))
                so = plsc.load_gather(smt, [fi])
                plsc.store_scatter(smt, [fi], so + xc)
            co = plsc.load_gather(cnt, [ci16])
            plsc.store_scatter(cnt, [ci16], co + 1.0)
        return 0
    lax.fori_loop(0, 20, grpn, 0)

    pltpu.sync_copy(mxt, mx_hbm.at[wid])
    pltpu.sync_copy(smt, sm_hbm.at[wid])
    pltpu.sync_copy(cnt, cn_hbm.at[wid])


def _pool_sc(xf, tf):
    kfn = pl.kernel(
        _pool_body,
        out_type=[jax.ShapeDtypeStruct((NT, NROW * 80), _f32),
                  jax.ShapeDtypeStruct((NT, NROW * 80), _f32),
                  jax.ShapeDtypeStruct((NT, NROW * 16), _f32)],
        mesh=_mesh(),
        compiler_params=_SC_PARAMS,
        scratch_types=[
            pltpu.VMEM((NROW * 80,), _f32),
            pltpu.VMEM((NROW * 80,), _f32),
            pltpu.VMEM((NROW * 16,), _f32),
            pltpu.VMEM((313 * 80,), _f32),
            pltpu.VMEM((320,), _f32),
        ],
    )
    return kfn(xf, tf)


def _proj_kernel(x_ref, w_ref, as_ref, ad_ref, h_ref, asd_ref, add_ref):
    h = jnp.dot(x_ref[...], w_ref[...], preferred_element_type=_f32)
    h_ref[...] = h
    asd_ref[...] = jnp.dot(h, as_ref[...], preferred_element_type=_f32)
    add_ref[...] = jnp.dot(h, ad_ref[...], preferred_element_type=_f32)


def _proj(x, wt, ast, adt):
    k = x.shape[1]
    return pl.pallas_call(
        _proj_kernel,
        grid=(8,),
        in_specs=[
            pl.BlockSpec((NPAD // 8, k), lambda i: (i, 0)),
            pl.BlockSpec((k, 80), lambda i: (0, 0)),
            pl.BlockSpec((80, 16), lambda i: (0, 0)),
            pl.BlockSpec((80, 16), lambda i: (0, 0)),
        ],
        out_specs=[
            pl.BlockSpec((NPAD // 8, 80), lambda i: (i, 0)),
            pl.BlockSpec((NPAD // 8, 16), lambda i: (i, 0)),
            pl.BlockSpec((NPAD // 8, 16), lambda i: (i, 0)),
        ],
        out_shape=[jax.ShapeDtypeStruct((NPAD, 80), _f32),
                   jax.ShapeDtypeStruct((NPAD, 16), _f32),
                   jax.ShapeDtypeStruct((NPAD, 16), _f32)],
    )(x, wt, ast, adt)


def _merge_kernel(mx_ref, sm_ref, cn_ref, mxo_ref, avo_ref):
    mx = jnp.max(mx_ref[...], axis=0)
    sm = jnp.sum(sm_ref[...], axis=0)
    cn = jnp.sum(cn_ref[...], axis=0)
    mxo_ref[...] = jnp.where(cn > 0.0, mx, 0.0)
    avo_ref[...] = sm / jnp.maximum(cn, 1.0)


def _merge(mx4, sm4, cn4):
    return pl.pallas_call(
        _merge_kernel,
        out_shape=[jax.ShapeDtypeStruct((B, SIZE * 80), _f32),
                   jax.ShapeDtypeStruct((B, SIZE * 80), _f32)],
    )(mx4, sm4, cn4)


def _lrelu(x):
    return jax.nn.leaky_relu(x, 0.01)


def _conv_kernel(ntap, lout, xe_ref, xo_ref, we_ref, wo_ref, b_ref, o_ref):
    acc = jnp.zeros((o_ref.shape[1], lout), _f32)
    xe = xe_ref[0]
    xo = xo_ref[0]
    for a in range(ntap):
        acc = acc + jnp.dot(we_ref[a], xe[:, a:a + lout],
                            preferred_element_type=_f32)
        acc = acc + jnp.dot(wo_ref[a], xo[:, a:a + lout],
                            preferred_element_type=_f32)
    o_ref[0] = _lrelu(acc + b_ref[...])


def _conv(xe, xo, we, wo, bvec, lout):
    cin = xe.shape[1]
    lin = xe.shape[2]
    cout = we.shape[1]
    return pl.pallas_call(
        functools.partial(_conv_kernel, 5, lout),
        grid=(B,),
        in_specs=[
            pl.BlockSpec((1, cin, lin), lambda i: (i, 0, 0)),
            pl.BlockSpec((1, cin, lin), lambda i: (i, 0, 0)),
            pl.BlockSpec((5, cout, cin), lambda i: (0, 0, 0)),
            pl.BlockSpec((5, cout, cin), lambda i: (0, 0, 0)),
            pl.BlockSpec((cout, 1), lambda i: (0, 0)),
        ],
        out_specs=pl.BlockSpec((1, cout, lout), lambda i: (i, 0, 0)),
        out_shape=jax.ShapeDtypeStruct((B, cout, lout), _f32),
    )(xe, xo, we, wo, bvec)


def _head_kernel(oc_ref, *refs):
    ws = refs[:-1]
    out_ref = refs[-1]
    oc = oc_ref[...]
    mx = jnp.max(oc, axis=-1)
    mean = jnp.sum(oc, axis=-1) * 0.1
    o = jnp.concatenate([mx, mean], axis=1)

    def dense(v, i):
        return jnp.dot(v, ws[2 * i][...], preferred_element_type=_f32) \
            + ws[2 * i + 1][...]

    for i in range(3):
        o = _lrelu(dense(o, i))
    chi = _lrelu(dense(o, 3))
    chi = _lrelu(dense(chi, 4))
    chi = jnp.tanh(dense(chi, 5))
    rp = _lrelu(dense(o, 6))
    rp = _lrelu(dense(rp, 7))
    rp = dense(rp, 8)
    t0 = _lrelu(dense(o, 9))
    t0 = _lrelu(dense(t0, 10))
    t0 = dense(t0, 11)
    out_ref[...] = jnp.concatenate([chi, rp, t0], axis=1)


def _head(oc, params):
    names = ['d1', 'd2', 'd3', 'chi1', 'chi2', 'chi3',
             'rp1', 'rp2', 'rp3', 't01', 't02', 't03']
    args = [oc]
    for n in names:
        args.append(params[n]['W'])
        args.append(params[n]['b'].reshape(1, -1))
    return pl.pallas_call(
        _head_kernel,
        out_shape=jax.ShapeDtypeStruct((B, 3), _f32),
    )(*args)


def _gat_weight_prep(p, first):
    w = p['W']
    if first:
        wt = jnp.zeros((8, 80), _f32).at[:5, :].set(w[:, _TMAP])
    else:
        wt = w[_TMAP][:, _TMAP]
    h_, c_ = p['att_src'].shape
    rows = np.arange(80)
    heads = rows // 5

    def blockdiag(att):
        m = jnp.zeros((80, 16), _f32)
        return m.at[rows, heads].set(att.reshape(-1))

    ast = blockdiag(p['att_src'])[_TMAP]
    adt = blockdiag(p['att_dst'])[_TMAP]
    aem = (p['We'].reshape(2, h_, c_) * p['att_e'][None]).sum(-1)
    mef = aem.reshape(32).astype(_f32)
    biasf = p['bias'][_TMAP].astype(_f32)
    return wt.astype(_f32), ast.astype(_f32), adt.astype(_f32), mef, biasf


def _conv_weight_prep(p):
    w = p['W']
    we = jnp.stack([w[:, :, 2 * a] for a in range(5)])
    wo = jnp.stack([w[:, :, 2 * a + 1] for a in range(5)])
    return we.astype(_f32), wo.astype(_f32), p['b'].reshape(-1, 1).astype(_f32)


def _deinterleave(x, lpad):
    xp = jnp.pad(x, ((0, 0), (0, 0), (1, lpad)))
    xr = xp.reshape(x.shape[0], x.shape[1], -1, 2)
    return xr[..., 0], xr[..., 1]


def kernel(nodes, edges, edge_v, batching, params):
    src = edges[:, 0].astype(_i32)
    dst = edges[:, 1].astype(_i32)
    ev0 = edge_v[:, 0].astype(_f32)
    ev1 = edge_v[:, 1].astype(_f32)
    timing_f = jnp.pad(nodes[:, 2], (0, NPAD - N_NODES)).astype(_f32)

    queues = _bucket(src, dst, ev0, ev1)

    x = jnp.zeros((NPAD, 8), _f32).at[:N_NODES, :5].set(nodes)
    layer_names = ['g1', 'g2', 'g3', 'gs']
    for li, name in enumerate(layer_names):
        wt, ast, adt, mef, biasf = _gat_weight_prep(params[name], li == 0)
        h2d, asd2d, add2d = _proj(x, wt, ast, adt)
        outf = _gat_sc(h2d, asd2d, add2d, mef, biasf, queues,
                       masked=(name == 'gs'))
        x = outf.reshape(NPAD, 80)

    mxp, smp, cnp = _pool_sc(outf, timing_f)

    def arrange(t, w):
        return t.reshape(B, 4, NROW, w)[:, :, :SIZE, :] \
            .transpose(1, 0, 2, 3).reshape(4, B, SIZE * w)

    mx4 = arrange(mxp, 80)
    sm4 = arrange(smp, 80)
    cn16 = arrange(cnp, 16).reshape(4, B, SIZE, 16)
    cn4 = jnp.broadcast_to(cn16[:, :, :, :1], (4, B, SIZE, 80)) \
        .reshape(4, B, SIZE * 80)
    mx8, av8 = _merge(mx4, sm4, cn4)

    def to_conv(t):
        return t.reshape(B, SIZE, 80)[:, :, _NTMAP].reshape(B, 80, SIZE)

    o = jnp.concatenate([to_conv(mx8), to_conv(av8)], axis=1)

    we1, wo1, b1 = _conv_weight_prep(params['conv1'])
    we2, wo2, b2 = _conv_weight_prep(params['conv2'])
    we3, wo3, b3 = _conv_weight_prep(params['conv3'])
    xe, xo = _deinterleave(o, 1)
    o1 = _conv(xe, xo, we1, wo1, b1, 247)
    xe, xo = _deinterleave(o1, 2)
    o2 = _conv(xe, xo, we2, wo2, b2, 120)
    xe, xo = _deinterleave(o2, 1)
    o3 = _conv(xe, xo, we3, wo3, b3, 57)

    oc = o3[:, :, :50].reshape(B, 16, 5, 10).reshape(B, 80, 10)
    return _head(oc, params)


# speedup vs baseline: 1.0298x; 1.0298x over previous
import functools

import jax
import jax.numpy as jnp
import numpy as np
from jax import lax
from jax.experimental import pallas as pl
from jax.experimental.pallas import tpu as pltpu
from jax.experimental.pallas import tpu_sc as plsc

N_NODES = 10000
N_EDGES = 160000
B = 8
HEADS = 16
SIZE = 500
NPAD = 10240
NT = 32
NPT = 320
QCAP = 12288
QW = QCAP * 4
CH = 128
SCH = 4000
NEG = -1e30
SENT = float(NPAD - 1)

_SC_PARAMS = pltpu.CompilerParams(
    use_tc_tiling_on_sc=False, needs_layout_passes=False)

_f32 = jnp.float32
_i32 = jnp.int32

_TMAP = np.array([hd * 5 + ch for ch in range(5) for hd in range(16)])
_NTMAP = np.array([ch * 16 + hd for hd in range(16) for ch in range(5)])


def _mesh():
    return plsc.VectorSubcoreMesh(core_axis_name="c", subcore_axis_name="s")


def _prefix16(csv, iota):
    for sh in (1, 2, 4, 8):
        sht = csv[jnp.maximum(iota - sh, 0)]
        csv = csv + jnp.where(iota >= sh, sht, 0.0)
    return csv


def _bucket_body(src_hbm, dst_hbm, e0_hbm, e1_hbm,
                 q_hbm, cnt_hbm,
                 sbuf, dbuf, e0b, e1b, qw, ob, sq):
    wid = lax.axis_index("s") * 2 + lax.axis_index("c")
    iota = lax.broadcasted_iota(_i32, (16,), 0)
    basef = (wid * NPT).astype(_f32)
    nchunks = N_EDGES // SCH
    sentv = jnp.full((16,), SENT, _f32)
    basev = jnp.full((16,), 0.0, _f32) + basef
    zerov = jnp.zeros((16,), _f32)

    def initq(i, _):
        fld = (i >> 3) & 3
        v = jnp.where(fld == 0, sentv, jnp.where(fld == 1, basev, zerov))
        qw[pl.ds(i * 16, 16)] = v
        return 0
    lax.fori_loop(0, QW // 16, initq, 0)

    widv = jnp.full((16,), 0, _i32) + wid

    def fire(c, par):
        off = c * SCH
        hof = par * SCH
        pltpu.async_copy(src_hbm.at[pl.ds(off, SCH)],
                         sbuf.at[pl.ds(hof, SCH)], sq)
        pltpu.async_copy(dst_hbm.at[pl.ds(off, SCH)],
                         dbuf.at[pl.ds(hof, SCH)], sq)
        pltpu.async_copy(e0_hbm.at[pl.ds(off, SCH)],
                         e0b.at[pl.ds(hof, SCH)], sq)
        pltpu.async_copy(e1_hbm.at[pl.ds(off, SCH)],
                         e1b.at[pl.ds(hof, SCH)], sq)

    def drain(c, par):
        off = c * SCH
        hof = par * SCH
        pltpu.make_async_copy(src_hbm.at[pl.ds(off, SCH)],
                              sbuf.at[pl.ds(hof, SCH)], sq).wait()
        pltpu.make_async_copy(dst_hbm.at[pl.ds(off, SCH)],
                              dbuf.at[pl.ds(hof, SCH)], sq).wait()
        pltpu.make_async_copy(e0_hbm.at[pl.ds(off, SCH)],
                              e0b.at[pl.ds(hof, SCH)], sq).wait()
        pltpu.make_async_copy(e1_hbm.at[pl.ds(off, SCH)],
                              e1b.at[pl.ds(hof, SCH)], sq).wait()

    fire(0, 0)

    def chunk(c, cnt):
        par = c & 1
        drain(c, par)

        @pl.when(c + 1 < nchunks)
        def _():
            fire(c + 1, 1 - par)

        hof = par * SCH

        def grp(g, cnt):
            dv = dbuf[pl.ds(hof + g * 16, 16)]
            bkt = (dv * 52429) >> 24
            own = bkt == widv
            mf = jnp.where(own, 1.0, 0.0)
            cs = _prefix16(mf, iota)
            posf = cnt + (cs - mf)
            posf = jnp.minimum(posf, float(QCAP - CH - 16))
            pos = posf.astype(_i32)
            qpos = ((pos >> 7) * 512) + (pos & 127)
            trash = (QCAP // CH - 1) * 512 + iota
            qpos = jnp.where(own, qpos, trash)
            sv = sbuf[pl.ds(hof + g * 16, 16)].astype(_f32)
            plsc.store_scatter(qw, [qpos], sv)
            plsc.store_scatter(qw, [qpos + 128], dv.astype(_f32))
            plsc.store_scatter(qw, [qpos + 256], e0b[pl.ds(hof + g * 16, 16)])
            plsc.store_scatter(qw, [qpos + 384], e1b[pl.ds(hof + g * 16, 16)])
            return cnt + cs[15]
        return lax.fori_loop(0, SCH // 16, grp, cnt)

    cnt = lax.fori_loop(0, nchunks, chunk, jnp.float32(0.0))
    cnt = jnp.minimum(cnt, float(QCAP - CH - 16))
    toff = (QCAP // CH - 1) * 512

    def fixt(i, _):
        fld = i >> 3
        v = jnp.where(fld == 0, sentv, jnp.where(fld == 1, basev, zerov))
        qw[pl.ds(toff + i * 16, 16)] = v
        return 0
    lax.fori_loop(0, 512 // 16, fixt, 0)

    ob[...] = jnp.full((16,), 0.0, _f32) + cnt
    pltpu.sync_copy(ob, cnt_hbm.at[wid])
    pltpu.sync_copy(qw, q_hbm.at[wid])


def _bucket(src, dst, ev0, ev1):
    kfn = pl.kernel(
        _bucket_body,
        out_type=[jax.ShapeDtypeStruct((NT, QW), _f32),
                  jax.ShapeDtypeStruct((NT, 16), _f32)],
        mesh=_mesh(),
        compiler_params=_SC_PARAMS,
        scratch_types=[
            pltpu.VMEM((2 * SCH,), _i32), pltpu.VMEM((2 * SCH,), _i32),
            pltpu.VMEM((2 * SCH,), _f32), pltpu.VMEM((2 * SCH,), _f32),
            pltpu.VMEM((QW,), _f32),
            pltpu.VMEM((16,), _f32),
            pltpu.SemaphoreType.DMA,
        ],
    )
    return kfn(src, dst, ev0, ev1)


def _gat_body(masked, h2d, asd2d, add2d, mef, biasf,
              q_hbm, cnt_hbm, out_hbm,
              m_t, den_t, acc_t, addl, asdl, hl,
              qb, idxc, a16, h80, meb, biasb, cb,
              sq, s1, s2):
    wid = lax.axis_index("s") * 2 + lax.axis_index("c")
    iota = lax.broadcasted_iota(_i32, (16,), 0)
    base = wid * NPT
    basef = jnp.full((16,), 0.0, _f32) + base.astype(_f32)

    pltpu.sync_copy(cnt_hbm.at[wid], cb)
    nmy = cb[...][0].astype(_i32)
    pltpu.sync_copy(add2d.at[pl.ds(base, NPT)], addl)
    pltpu.sync_copy(asd2d.at[pl.ds(base, NPT)], asdl)
    pltpu.sync_copy(h2d.at[pl.ds(base, NPT)], hl)
    pltpu.sync_copy(mef, meb)
    pltpu.sync_copy(biasf, biasb)
    me0 = meb[pl.ds(0, 16)]
    me1 = meb[pl.ds(16, 16)]

    negv = jnp.full((16,), NEG, _f32)
    zerov = jnp.zeros((16,), _f32)

    def init1(i, _):
        m_t[pl.ds(i * 16, 16)] = negv
        den_t[pl.ds(i * 16, 16)] = zerov
        return 0
    lax.fori_loop(0, NPT, init1, 0)

    def init2(i, _):
        acc_t[pl.ds(i * 16, 16)] = zerov
        return 0
    lax.fori_loop(0, NPT * 5, init2, 0)

    nch = (nmy + (CH - 1)) // CH

    def fire_q(c, par):
        pltpu.async_copy(q_hbm.at[wid, pl.ds(c * 512, 512)],
                         qb.at[pl.ds(par * 512, 512)], sq)

    def drain_q(c, par):
        pltpu.make_async_copy(q_hbm.at[wid, pl.ds(c * 512, 512)],
                              qb.at[pl.ds(par * 512, 512)], sq).wait()

    def cvt_fire(c, par):
        qoff = par * 512

        def cvt(g, _2):
            idxc[pl.ds(par * CH + g * 16, 16)] = \
                qb[pl.ds(qoff + g * 16, 16)].astype(_i32)
            return 0
        lax.fori_loop(0, CH // 16, cvt, 0)
        pltpu.async_copy(asd2d.at[idxc.at[pl.ds(par * CH, CH)]],
                         a16.at[pl.ds(par * CH, CH)], s1)
        pltpu.async_copy(h2d.at[idxc.at[pl.ds(par * CH, CH)]],
                         h80.at[pl.ds(par * CH, CH)], s2)

    def drain_ind(par):
        pltpu.make_async_copy(asd2d.at[idxc.at[pl.ds(par * CH, CH)]],
                              a16.at[pl.ds(par * CH, CH)], s1).wait()
        pltpu.make_async_copy(h2d.at[idxc.at[pl.ds(par * CH, CH)]],
                              h80.at[pl.ds(par * CH, CH)], s2).wait()

    @pl.when(nch > 0)
    def _():
        fire_q(0, 0)
        drain_q(0, 0)
        cvt_fire(0, 0)

    @pl.when(nch > 1)
    def _():
        fire_q(1, 1)

    def chunk(c, _):
        par = c & 1
        parn = 1 - par

        @pl.when(c + 1 < nch)
        def _():
            drain_q(c + 1, parn)
            cvt_fire(c + 1, parn)

        drain_ind(par)
        qoff = par * 512

        def grp(g, _2):
            dlv = qb[pl.ds(qoff + 128 + g * 16, 16)] - basef
            e0v = qb[pl.ds(qoff + 256 + g * 16, 16)]
            e1v = qb[pl.ds(qoff + 384 + g * 16, 16)]
            sv = qb[pl.ds(qoff + g * 16, 16)]
            ebv = jnp.where(sv == SENT, NEG, 0.0)
            for j in range(16):
                jf = jnp.full((16,), j, _i32)
                dl_i = dlv[j].astype(_i32)
                e0j = e0v[jf]
                e1j = e1v[jf]
                ebj = ebv[jf]
                fidx = dl_i * 16 + iota
                ae = e0j * me0 + e1j * me1
                if masked:
                    ae = ae + jnp.where(e0j > 1.0, 0.0, NEG)
                ae = ae + ebj
                asdr = a16[par * CH + g * 16 + j, :]
                addr = addl[dl_i, :]
                z = asdr + addr + ae
                z = jnp.maximum(z, 0.2 * z)
                mo = plsc.load_gather(m_t, [fidx])
                do = plsc.load_gather(den_t, [fidx])
                mn = jnp.maximum(mo, z)
                sc = jnp.exp(mo - mn)
                ex = jnp.exp(z - mn)
                plsc.store_scatter(m_t, [fidx], mn)
                plsc.store_scatter(den_t, [fidx], do * sc + ex)
                fidx80 = dl_i * 80 + iota
                for cc in range(5):
                    fi = fidx80 + 16 * cc
                    ao = plsc.load_gather(acc_t, [fi])
                    hr = h80[par * CH + g * 16 + j, pl.ds(16 * cc, 16)]
                    plsc.store_scatter(acc_t, [fi], ao * sc + ex * hr)
            return 0
        lax.fori_loop(0, CH // 16, grp, 0)

        @pl.when(c + 2 < nch)
        def _():
            fire_q(c + 2, par)
        return 0
    lax.fori_loop(0, nch, chunk, 0)

    def fin(i, _):
        asdr = asdl[i, :]
        addr = addl[i, :]
        zs = asdr + addr
        zs = jnp.maximum(zs, 0.2 * zs)
        mo = m_t[pl.ds(i * 16, 16)]
        do = den_t[pl.ds(i * 16, 16)]
        mn = jnp.maximum(mo, zs)
        sc = jnp.exp(mo - mn)
        exs = jnp.exp(zs - mn)
        den = do * sc + exs
        inv = 1.0 / (den + 1e-16)
        for cc in range(5):
            ac = acc_t[pl.ds(i * 80 + 16 * cc, 16)]
            hc = hl[i, pl.ds(16 * cc, 16)]
            acc_t[pl.ds(i * 80 + 16 * cc, 16)] = (
                (ac * sc + exs * hc) * inv + biasb[pl.ds(16 * cc, 16)])
        return 0
    lax.fori_loop(0, NPT, fin, 0)

    pltpu.sync_copy(acc_t, out_hbm.at[pl.ds(base * 80, NPT * 80)])


def _gat_sc(h2d, asd2d, add2d, mef, biasf, queues, masked):
    qu, cnt = queues
    kfn = pl.kernel(
        functools.partial(_gat_body, masked),
        out_type=jax.ShapeDtypeStruct((NPAD * 80,), _f32),
        mesh=_mesh(),
        compiler_params=_SC_PARAMS,
        scratch_types=[
            pltpu.VMEM((NPT * 16,), _f32),
            pltpu.VMEM((NPT * 16,), _f32),
            pltpu.VMEM((NPT * 80,), _f32),
            pltpu.VMEM((NPT, 16), _f32),
            pltpu.VMEM((NPT, 16), _f32),
            pltpu.VMEM((NPT, 80), _f32),
            pltpu.VMEM((2 * 512,), _f32),
            pltpu.VMEM((2 * CH,), _i32),
            pltpu.VMEM((2 * CH, 16), _f32),
            pltpu.VMEM((2 * CH, 80), _f32),
            pltpu.VMEM((32,), _f32),
            pltpu.VMEM((80,), _f32),
            pltpu.VMEM((16,), _f32),
            pltpu.SemaphoreType.DMA,
            pltpu.SemaphoreType.DMA,
            pltpu.SemaphoreType.DMA,
        ],
    )
    return kfn(h2d, asd2d, add2d, mef, biasf, qu, cnt)


NROW = SIZE + 1


def _pool_body(xf_hbm, tf_hbm, mx_hbm, sm_hbm, cn_hbm,
               mxt, smt, cnt, xb, tb):
    wid = lax.axis_index("s") * 2 + lax.axis_index("c")
    iota = lax.broadcasted_iota(_i32, (16,), 0)
    b = wid >> 2
    r = wid & 3
    start = b * 1250 + r * 313
    nmine = jnp.where(r == 3, 311, 313)
    sstart = (start // 8) * 8
    doff = start - sstart

    lowv = jnp.full((16,), -3.0e38, _f32)
    zerov = jnp.zeros((16,), _f32)

    def init1(i, _):
        mxt[pl.ds(i * 16, 16)] = lowv
        smt[pl.ds(i * 16, 16)] = zerov
        return 0
    lax.fori_loop(0, NROW * 5, init1, 0)

    def init2(i, _):
        cnt[pl.ds(i * 16, 16)] = zerov
        return 0
    lax.fori_loop(0, NROW, init2, 0)

    pltpu.sync_copy(xf_hbm.at[pl.ds(start * 80, 313 * 80)], xb)
    pltpu.sync_copy(tf_hbm.at[pl.ds(sstart, 320)], tb)

    def grpn(g, _):
        gl = g * 16 + iota
        tv = tb[pl.ds(g * 16 + doff, 16)]
        tv = jnp.where(gl < nmine, tv, float(SIZE))
        for j in range(16):
            jf = jnp.full((16,), j, _i32)
            tj = tv[jf]
            ci80 = (tj * 80.0).astype(_i32) + iota
            ci16 = (tj * 16.0).astype(_i32) + iota
            node = g * 16 + j
            for cc in range(5):
                fi = ci80 + 16 * cc
                xc = xb[pl.ds(node * 80 + 16 * cc, 16)]
                mo = plsc.load_gather(mxt, [fi])
                plsc.store_scatter(mxt, [fi], jnp.maximum(mo, xc))
                so = plsc.load_gather(smt, [fi])
                plsc.store_scatter(smt, [fi], so + xc)
            co = plsc.load_gather(cnt, [ci16])
            plsc.store_scatter(cnt, [ci16], co + 1.0)
        return 0
    lax.fori_loop(0, 20, grpn, 0)

    pltpu.sync_copy(mxt, mx_hbm.at[wid])
    pltpu.sync_copy(smt, sm_hbm.at[wid])
    pltpu.sync_copy(cnt, cn_hbm.at[wid])


def _pool_sc(xf, tf):
    kfn = pl.kernel(
        _pool_body,
        out_type=[jax.ShapeDtypeStruct((NT, NROW * 80), _f32),
                  jax.ShapeDtypeStruct((NT, NROW * 80), _f32),
                  jax.ShapeDtypeStruct((NT, NROW * 16), _f32)],
        mesh=_mesh(),
        compiler_params=_SC_PARAMS,
        scratch_types=[
            pltpu.VMEM((NROW * 80,), _f32),
            pltpu.VMEM((NROW * 80,), _f32),
            pltpu.VMEM((NROW * 16,), _f32),
            pltpu.VMEM((313 * 80,), _f32),
            pltpu.VMEM((320,), _f32),
        ],
    )
    return kfn(xf, tf)


def _proj_kernel(x_ref, w_ref, as_ref, ad_ref, h_ref, asd_ref, add_ref):
    h = jnp.dot(x_ref[...], w_ref[...], preferred_element_type=_f32)
    h_ref[...] = h
    asd_ref[...] = jnp.dot(h, as_ref[...], preferred_element_type=_f32)
    add_ref[...] = jnp.dot(h, ad_ref[...], preferred_element_type=_f32)


def _proj(x, wt, ast, adt):
    k = x.shape[1]
    return pl.pallas_call(
        _proj_kernel,
        grid=(8,),
        in_specs=[
            pl.BlockSpec((NPAD // 8, k), lambda i: (i, 0)),
            pl.BlockSpec((k, 80), lambda i: (0, 0)),
            pl.BlockSpec((80, 16), lambda i: (0, 0)),
            pl.BlockSpec((80, 16), lambda i: (0, 0)),
        ],
        out_specs=[
            pl.BlockSpec((NPAD // 8, 80), lambda i: (i, 0)),
            pl.BlockSpec((NPAD // 8, 16), lambda i: (i, 0)),
            pl.BlockSpec((NPAD // 8, 16), lambda i: (i, 0)),
        ],
        out_shape=[jax.ShapeDtypeStruct((NPAD, 80), _f32),
                   jax.ShapeDtypeStruct((NPAD, 16), _f32),
                   jax.ShapeDtypeStruct((NPAD, 16), _f32)],
    )(x, wt, ast, adt)


def _merge_kernel(mx_ref, sm_ref, cn_ref, mxo_ref, avo_ref):
    mx = jnp.max(mx_ref[...], axis=0)
    sm = jnp.sum(sm_ref[...], axis=0)
    cn = jnp.sum(cn_ref[...], axis=0)
    mxo_ref[...] = jnp.where(cn > 0.0, mx, 0.0)
    avo_ref[...] = sm / jnp.maximum(cn, 1.0)


def _merge(mx4, sm4, cn4):
    return pl.pallas_call(
        _merge_kernel,
        out_shape=[jax.ShapeDtypeStruct((B, SIZE * 80), _f32),
                   jax.ShapeDtypeStruct((B, SIZE * 80), _f32)],
    )(mx4, sm4, cn4)


def _lrelu(x):
    return jax.nn.leaky_relu(x, 0.01)


def _conv_kernel(ntap, lout, xe_ref, xo_ref, we_ref, wo_ref, b_ref, o_ref):
    acc = jnp.zeros((o_ref.shape[1], lout), _f32)
    xe = xe_ref[0]
    xo = xo_ref[0]
    for a in range(ntap):
        acc = acc + jnp.dot(we_ref[a], xe[:, a:a + lout],
                            preferred_element_type=_f32)
        acc = acc + jnp.dot(wo_ref[a], xo[:, a:a + lout],
                            preferred_element_type=_f32)
    o_ref[0] = _lrelu(acc + b_ref[...])


def _conv(xe, xo, we, wo, bvec, lout):
    cin = xe.shape[1]
    lin = xe.shape[2]
    cout = we.shape[1]
    return pl.pallas_call(
        functools.partial(_conv_kernel, 5, lout),
        grid=(B,),
        in_specs=[
            pl.BlockSpec((1, cin, lin), lambda i: (i, 0, 0)),
            pl.BlockSpec((1, cin, lin), lambda i: (i, 0, 0)),
            pl.BlockSpec((5, cout, cin), lambda i: (0, 0, 0)),
            pl.BlockSpec((5, cout, cin), lambda i: (0, 0, 0)),
            pl.BlockSpec((cout, 1), lambda i: (0, 0)),
        ],
        out_specs=pl.BlockSpec((1, cout, lout), lambda i: (i, 0, 0)),
        out_shape=jax.ShapeDtypeStruct((B, cout, lout), _f32),
    )(xe, xo, we, wo, bvec)


def _head_kernel(oc_ref, *refs):
    ws = refs[:-1]
    out_ref = refs[-1]
    oc = oc_ref[...]
    mx = jnp.max(oc, axis=-1)
    mean = jnp.sum(oc, axis=-1) * 0.1
    o = jnp.concatenate([mx, mean], axis=1)

    def dense(v, i):
        return jnp.dot(v, ws[2 * i][...], preferred_element_type=_f32) \
            + ws[2 * i + 1][...]

    for i in range(3):
        o = _lrelu(dense(o, i))
    chi = _lrelu(dense(o, 3))
    chi = _lrelu(dense(chi, 4))
    chi = jnp.tanh(dense(chi, 5))
    rp = _lrelu(dense(o, 6))
    rp = _lrelu(dense(rp, 7))
    rp = dense(rp, 8)
    t0 = _lrelu(dense(o, 9))
    t0 = _lrelu(dense(t0, 10))
    t0 = dense(t0, 11)
    out_ref[...] = jnp.concatenate([chi, rp, t0], axis=1)


def _head(oc, params):
    names = ['d1', 'd2', 'd3', 'chi1', 'chi2', 'chi3',
             'rp1', 'rp2', 'rp3', 't01', 't02', 't03']
    args = [oc]
    for n in names:
        args.append(params[n]['W'])
        args.append(params[n]['b'].reshape(1, -1))
    return pl.pallas_call(
        _head_kernel,
        out_shape=jax.ShapeDtypeStruct((B, 3), _f32),
    )(*args)


def _gat_weight_prep(p, first):
    w = p['W']
    if first:
        wt = jnp.zeros((8, 80), _f32).at[:5, :].set(w[:, _TMAP])
    else:
        wt = w[_TMAP][:, _TMAP]
    h_, c_ = p['att_src'].shape
    rows = np.arange(80)
    heads = rows // 5

    def blockdiag(att):
        m = jnp.zeros((80, 16), _f32)
        return m.at[rows, heads].set(att.reshape(-1))

    ast = blockdiag(p['att_src'])[_TMAP]
    adt = blockdiag(p['att_dst'])[_TMAP]
    aem = (p['We'].reshape(2, h_, c_) * p['att_e'][None]).sum(-1)
    mef = aem.reshape(32).astype(_f32)
    biasf = p['bias'][_TMAP].astype(_f32)
    return wt.astype(_f32), ast.astype(_f32), adt.astype(_f32), mef, biasf


def _conv_weight_prep(p):
    w = p['W']
    we = jnp.stack([w[:, :, 2 * a] for a in range(5)])
    wo = jnp.stack([w[:, :, 2 * a + 1] for a in range(5)])
    return we.astype(_f32), wo.astype(_f32), p['b'].reshape(-1, 1).astype(_f32)


def _deinterleave(x, lpad):
    xp = jnp.pad(x, ((0, 0), (0, 0), (1, lpad)))
    xr = xp.reshape(x.shape[0], x.shape[1], -1, 2)
    return xr[..., 0], xr[..., 1]


def kernel(nodes, edges, edge_v, batching, params):
    src = edges[:, 0].astype(_i32)
    dst = edges[:, 1].astype(_i32)
    ev0 = edge_v[:, 0].astype(_f32)
    ev1 = edge_v[:, 1].astype(_f32)
    timing_f = jnp.pad(nodes[:, 2], (0, NPAD - N_NODES)).astype(_f32)

    queues = _bucket(src, dst, ev0, ev1)

    x = jnp.zeros((NPAD, 8), _f32).at[:N_NODES, :5].set(nodes)
    layer_names = ['g1', 'g2', 'g3', 'gs']
    for li, name in enumerate(layer_names):
        wt, ast, adt, mef, biasf = _gat_weight_prep(params[name], li == 0)
        h2d, asd2d, add2d = _proj(x, wt, ast, adt)
        outf = _gat_sc(h2d, asd2d, add2d, mef, biasf, queues,
                       masked=(name == 'gs'))
        x = outf.reshape(NPAD, 80)

    mxp, smp, cnp = _pool_sc(outf, timing_f)

    def arrange(t, w):
        return t.reshape(B, 4, NROW, w)[:, :, :SIZE, :] \
            .transpose(1, 0, 2, 3).reshape(4, B, SIZE * w)

    mx4 = arrange(mxp, 80)
    sm4 = arrange(smp, 80)
    cn16 = arrange(cnp, 16).reshape(4, B, SIZE, 16)
    cn4 = jnp.broadcast_to(cn16[:, :, :, :1], (4, B, SIZE, 80)) \
        .reshape(4, B, SIZE * 80)
    mx8, av8 = _merge(mx4, sm4, cn4)

    def to_conv(t):
        return t.reshape(B, SIZE, 80)[:, :, _NTMAP].reshape(B, 80, SIZE)

    o = jnp.concatenate([to_conv(mx8), to_conv(av8)], axis=1)

    we1, wo1, b1 = _conv_weight_prep(params['conv1'])
    we2, wo2, b2 = _conv_weight_prep(params['conv2'])
    we3, wo3, b3 = _conv_weight_prep(params['conv3'])
    xe, xo = _deinterleave(o, 1)
    o1 = _conv(xe, xo, we1, wo1, b1, 247)
    xe, xo = _deinterleave(o1, 2)
    o2 = _conv(xe, xo, we2, wo2, b2, 120)
    xe, xo = _deinterleave(o2, 1)
    o3 = _conv(xe, xo, we3, wo3, b3, 57)

    oc = o3[:, :, :50].reshape(B, 16, 5, 10).reshape(B, 80, 10)
    return _head(oc, params)
